# baseline (speedup 1.0000x reference)
import functools

import jax
import jax.numpy as jnp
from jax import lax
from jax.experimental import pallas as pl
from jax.experimental.pallas import tpu as pltpu
from jax.experimental.pallas import tpu_sc as plsc

_CHUNK = 128


@functools.lru_cache(maxsize=None)
def _build(n_user, n_item, batch, embed):
    info = plsc.get_sparse_core_info()
    num_cores = info.num_cores
    num_workers = info.num_cores * info.num_subcores
    assert batch % (num_workers * _CHUNK) == 0
    b_per_w = batch // num_workers
    n_chunks = b_per_w // _CHUNK

    mesh = plsc.VectorSubcoreMesh(core_axis_name="c", subcore_axis_name="s")

    @functools.partial(
        pl.kernel,
        mesh=mesh,
        out_type=jax.ShapeDtypeStruct((2 * batch, embed), jnp.float32),
        compiler_params=pltpu.CompilerParams(use_tc_tiling_on_sc=False),
        scratch_types=[
            pltpu.VMEM((n_chunks, _CHUNK), jnp.int32),
            pltpu.VMEM((n_chunks, _CHUNK), jnp.int32),
            pltpu.VMEM((b_per_w, embed), jnp.float32),
            pltpu.VMEM((b_per_w, embed), jnp.float32),
            pltpu.SemaphoreType.DMA,
            pltpu.SemaphoreType.DMA,
            pltpu.SemaphoreType.DMA,
        ],
    )
    def run(user_hbm, item_hbm, idx_u_hbm, idx_i_hbm, out_hbm,
            idx_u_v, idx_i_v, rows_u, rows_i, usem, isem, wsem):
        wid = lax.axis_index("s") * num_cores + lax.axis_index("c")
        base = wid * b_per_w

        pltpu.sync_copy(idx_u_hbm.at[wid], idx_u_v)
        pltpu.sync_copy(idx_i_hbm.at[wid], idx_i_v)

        u_copies = [
            pltpu.async_copy(
                user_hbm.at[idx_u_v.at[c]],
                rows_u.at[pl.ds(c * _CHUNK, _CHUNK)],
                usem,
            )
            for c in range(n_chunks)
        ]
        i_copies = [
            pltpu.async_copy(
                item_hbm.at[idx_i_v.at[c]],
                rows_i.at[pl.ds(c * _CHUNK, _CHUNK)],
                isem,
            )
            for c in range(n_chunks)
        ]

        for cp in u_copies:
            cp.wait()
        w_u = pltpu.async_copy(rows_u, out_hbm.at[pl.ds(base, b_per_w)], wsem)
        for cp in i_copies:
            cp.wait()
        w_i = pltpu.async_copy(
            rows_i, out_hbm.at[pl.ds(batch + base, b_per_w)], wsem)
        w_u.wait()
        w_i.wait()

    def call(embed_user, embed_item, idx_user, idx_item):
        idx_u = idx_user.astype(jnp.int32).reshape(num_workers, n_chunks, _CHUNK)
        idx_i = idx_item.astype(jnp.int32).reshape(num_workers, n_chunks, _CHUNK)
        return run(embed_user, embed_item, idx_u, idx_i)

    return call


def kernel(embed_user, embed_item, idx_user, idx_item):
    n_user, embed = embed_user.shape
    n_item = embed_item.shape[0]
    batch = idx_user.shape[0]
    return _build(n_user, n_item, batch, embed)(
        embed_user, embed_item, idx_user, idx_item)
